# baseline (speedup 1.0000x reference)
import functools

import jax
import jax.numpy as jnp
from jax import lax
from jax.experimental import pallas as pl
from jax.experimental.pallas import tpu as pltpu
from jax.experimental.pallas import tpu_sc as plsc

N = 10000
E = 320000
D = 128
R = 4

NC = 2
NS = 16
NW = NC * NS
CH = 128
CW = 80
EW = CH * CW
E_PAD = NW * EW
ACC_ROWS = 10240
RPT = ACC_ROWS // NS
NPAD = 10016


def _dense_prep(x, W_self, b_self, W_rel, b_rel, wa, ba2):
    BM = 400
    G = N // BM

    def body(x_ref, ws_ref, bs_ref, wr_ref, br_ref, wa_ref, ba_ref,
             out0_ref, y_ref, a12_ref):
        xb = x_ref[...]
        out0_ref[...] = (
            jnp.dot(xb, ws_ref[...], preferred_element_type=jnp.float32)
            + bs_ref[...]
        )
        xb16 = xb.astype(jnp.bfloat16)
        for r in range(R):
            yr = (
                jnp.dot(xb16, wr_ref[r].astype(jnp.bfloat16),
                        preferred_element_type=jnp.float32)
                + br_ref[r]
            )
            bits = lax.bitcast_convert_type(
                yr.astype(jnp.bfloat16), jnp.uint16)
            lo = bits[:, : D // 2].astype(jnp.uint32)
            hi = bits[:, D // 2 :].astype(jnp.uint32)
            y_ref[r] = lax.bitcast_convert_type(lo | (hi << 16), jnp.int32)
        a12_ref[...] = (
            jnp.dot(xb16, wa_ref[...].astype(jnp.bfloat16),
                    preferred_element_type=jnp.float32)
            + ba_ref[...]
        )

    return pl.pallas_call(
        body,
        grid=(G,),
        in_specs=[
            pl.BlockSpec((BM, D), lambda i: (i, 0)),
            pl.BlockSpec((D, D), lambda i: (0, 0)),
            pl.BlockSpec((D,), lambda i: (0,)),
            pl.BlockSpec((R, D, D), lambda i: (0, 0, 0)),
            pl.BlockSpec((R, D), lambda i: (0, 0)),
            pl.BlockSpec((D, 2), lambda i: (0, 0)),
            pl.BlockSpec((1, 2), lambda i: (0, 0)),
        ],
        out_specs=[
            pl.BlockSpec((BM, D), lambda i: (i, 0)),
            pl.BlockSpec((R, BM, D // 2), lambda i: (0, i, 0)),
            pl.BlockSpec((BM, 2), lambda i: (i, 0)),
        ],
        out_shape=[
            jax.ShapeDtypeStruct((N, D), jnp.float32),
            jax.ShapeDtypeStruct((R, N, D // 2), jnp.int32),
            jax.ShapeDtypeStruct((N, 2), jnp.float32),
        ],
    )(x, W_self, b_self, W_rel, b_rel, wa, ba2)


SCN = 8
GRP = CW // SCN


def _sc_edge_agg(yflat, pk, srcp, typp, dstp2):
    mesh = plsc.VectorSubcoreMesh(core_axis_name="c", subcore_axis_name="s")

    @functools.partial(
        pl.kernel,
        out_type=jax.ShapeDtypeStruct((NC, ACC_ROWS, D), jnp.float32),
        mesh=mesh,
        compiler_params=pltpu.CompilerParams(
            needs_layout_passes=False, use_tc_tiling_on_sc=False),
        scratch_types=[
            pltpu.VMEM((NPAD,), jnp.int32),
            pltpu.VMEM((SCN, CH), jnp.int32),
            pltpu.VMEM((4 * SCN, CH // 4), jnp.int32),
            pltpu.VMEM((SCN, CH), jnp.float32),
            pltpu.VMEM((2 * SCN, CH // 2), jnp.int32),
            pltpu.VMEM((CH, D // 2), jnp.int32),
            pltpu.VMEM((CH, D // 2), jnp.int32),
            pltpu.VMEM((CH, D), jnp.float32),
            pltpu.VMEM_SHARED((ACC_ROWS, D), jnp.float32),
            [pltpu.SemaphoreType.DMA] * 8,
            pltpu.SemaphoreType.DMA,
            pltpu.SemaphoreType.DMA,
        ],
    )
    def k(y_hbm, pk_hbm, src_hbm, typ_hbm, dst2_hbm,
          part_hbm, pkv, srcv, ridv, attv, dstv2, rin0, rin1, rout, acc,
          semg, semsA, semsB):
        c = lax.axis_index("c")
        s = lax.axis_index("s")
        w = s * NC + c

        pltpu.sync_copy(pk_hbm, pkv)

        z16 = jnp.zeros((16,), jnp.float32)

        def zrow(i, carry):
            def zcol(q, carry2):
                rout[i, pl.ds(q * 16, 16)] = z16
                return carry2
            return lax.fori_loop(0, D // 16, zcol, carry)
        lax.fori_loop(0, CH, zrow, 0)
        for m in range(RPT // CH):
            pltpu.sync_copy(rout, acc.at[pl.ds(s * RPT + m * CH, CH)])
        plsc.subcore_barrier()

        hi_mask = jnp.full((16,), -65536, jnp.int32)

        def group(g, carry):
            gsl = pl.ds(g * SCN, SCN)
            g2sl = pl.ds(g * 2 * SCN, 2 * SCN)
            g4sl = pl.ds(g * 4 * SCN, 4 * SCN)
            pltpu.sync_copy(src_hbm.at[w, gsl], srcv)
            pltpu.sync_copy(typ_hbm.at[w, g4sl], ridv)
            pltpu.sync_copy(dst2_hbm.at[w, g2sl], dstv2)

            def att_row(j, carry1):
                def att_lane(q, carry2):
                    sl = pl.ds(q * 16, 16)
                    h2 = 2 * j + q // 4
                    sl2 = pl.ds((q % 4) * 16, 16)
                    h4 = 4 * j + q // 2
                    sl4 = pl.ds((q % 2) * 16, 16)
                    s16 = srcv[j, sl]
                    d16 = dstv2[h2, sl2]
                    t16 = ridv[h4, sl4]
                    g1 = plsc.load_gather(pkv, [s16])
                    g2 = plsc.load_gather(pkv, [d16])
                    a1f = plsc.bitcast(lax.shift_left(g1, 16), jnp.float32)
                    a2f = plsc.bitcast(g2 & hi_mask, jnp.float32)
                    z = a1f + a2f
                    attv[j, sl] = 1.0 / (1.0 + jnp.exp(-z))
                    ridv[h4, sl4] = t16 * N + s16
                    return carry2
                return lax.fori_loop(0, CH // 16, att_lane, carry1)

            bufs = (rin0, rin1)
            sems = (semsA, semsB)
            HC = CH // 2

            def unpack_half(j, h, rin):
                def us16(gg, carry2):
                    att16 = attv[j, pl.ds(h * HC + gg * 16, 16)]
                    for l in range(16):
                        a = att16[l]
                        e = h * HC + gg * 16 + l
                        for q in range(D // 32):
                            gv = rin[e, pl.ds(q * 16, 16)]
                            lo = plsc.bitcast(lax.shift_left(gv, 16),
                                              jnp.float32)
                            hi = plsc.bitcast(gv & hi_mask, jnp.float32)
                            rout[e, pl.ds(q * 16, 16)] = lo * a
                            rout[e, pl.ds(D // 2 + q * 16, 16)] = hi * a
                    return carry2
                lax.fori_loop(0, HC // 16, us16, 0)

            QC = CH // 4

            def gath(j, nb):
                return tuple(
                    pltpu.async_copy(
                        y_hbm.at[ridv.at[4 * j + p]],
                        bufs[nb].at[pl.ds(p * QC, QC)], semg[nb * 4 + p])
                    for p in range(4))

            att_row(0, 0)
            descs_g = [None] * SCN
            descs_s = [[None] * SCN, [None] * SCN]
            descs_g[0] = gath(0, 0)
            lax.fori_loop(1, SCN, att_row, 0)
            for j in range(SCN):
                b = j % 2
                if j + 1 < SCN:
                    descs_g[j + 1] = gath(j + 1, (j + 1) % 2)
                for h in range(2):
                    descs_g[j][2 * h].wait()
                    descs_g[j][2 * h + 1].wait()
                    if j >= 1:
                        descs_s[h][j - 1].wait()
                    unpack_half(j, h, bufs[b])
                    descs_s[h][j] = pltpu.async_copy(
                        rout.at[pl.ds(h * HC, HC)],
                        acc.at[dstv2.at[2 * j + h]], sems[h], add=True)
            descs_s[0][SCN - 1].wait()
            descs_s[1][SCN - 1].wait()
            return carry
        lax.fori_loop(0, GRP, group, 0)

        plsc.subcore_barrier()
        sl = pl.ds(s * RPT, RPT)
        pltpu.sync_copy(acc.at[sl], part_hbm.at[c, sl])

    return k(yflat, pk, srcp, typp, dstp2)


def _combine(out0, part):
    BM = 400
    G = N // BM

    def body(o0_ref, p_ref, o_ref):
        o_ref[...] = o0_ref[...] + p_ref[0] + p_ref[1]

    return pl.pallas_call(
        body,
        grid=(G,),
        in_specs=[
            pl.BlockSpec((BM, D), lambda i: (i, 0)),
            pl.BlockSpec((NC, BM, D), lambda i: (0, i, 0)),
        ],
        out_specs=pl.BlockSpec((BM, D), lambda i: (i, 0)),
        out_shape=jax.ShapeDtypeStruct((N, D), jnp.float32),
    )(out0, part)


def kernel(x, edge_index, edge_types, W_self, b_self, W_rel, b_rel, W_att, b_att):
    wa = jnp.stack([W_att[:D, 0], W_att[D:, 0]], axis=1)
    ba2 = jnp.stack([b_att[0], jnp.zeros((), jnp.float32)])[None, :]

    out0, y, a12 = _dense_prep(x, W_self, b_self, W_rel, b_rel, wa, ba2)
    yflat = y.reshape(R * N, D // 2)
    a1p = jnp.pad(a12[:, 0], (0, NPAD - N))
    a2p = jnp.pad(a12[:, 1], (0, NPAD - N))
    b1 = lax.bitcast_convert_type(a1p.astype(jnp.bfloat16), jnp.uint16)
    b2 = lax.bitcast_convert_type(a2p.astype(jnp.bfloat16), jnp.uint16)
    pk = lax.bitcast_convert_type(
        b1.astype(jnp.uint32) | (b2.astype(jnp.uint32) << 16), jnp.int32)

    pad = E_PAD - E
    idt = edge_index.dtype
    srcp = jnp.concatenate(
        [edge_index[0], jnp.zeros((pad,), idt)]).reshape(NW, CW, CH)
    dstp = jnp.concatenate(
        [edge_index[1], jnp.full((pad,), N, idt)]).reshape(NW, CW, CH)
    typp = jnp.concatenate(
        [edge_types, jnp.zeros((pad,), edge_types.dtype)]
    ).reshape(NW, 4 * CW, CH // 4)

    dstp2 = dstp.reshape(NW, 2 * CW, CH // 2)
    part = _sc_edge_agg(yflat, pk, srcp, typp, dstp2)
    return _combine(out0, part)


# speedup vs baseline: 1.0012x; 1.0012x over previous
import functools

import jax
import jax.numpy as jnp
from jax import lax
from jax.experimental import pallas as pl
from jax.experimental.pallas import tpu as pltpu
from jax.experimental.pallas import tpu_sc as plsc

N = 10000
E = 320000
D = 128
R = 4

NC = 2
NS = 16
NW = NC * NS
CH = 128
CW = 80
EW = CH * CW
E_PAD = NW * EW
ACC_ROWS = 10240
RPT = ACC_ROWS // NS
NPAD = 10016


def _dense_prep(x, W_self, b_self, W_rel, b_rel, wa, ba2):
    BM = 400
    G = N // BM

    def body(x_ref, ws_ref, bs_ref, wr_ref, br_ref, wa_ref, ba_ref,
             out0_ref, y_ref, a12_ref):
        xb = x_ref[...]
        out0_ref[...] = (
            jnp.dot(xb, ws_ref[...], preferred_element_type=jnp.float32)
            + bs_ref[...]
        )
        xb16 = xb.astype(jnp.bfloat16)
        for r in range(R):
            yr = (
                jnp.dot(xb16, wr_ref[r].astype(jnp.bfloat16),
                        preferred_element_type=jnp.float32)
                + br_ref[r]
            )
            bits = lax.bitcast_convert_type(
                yr.astype(jnp.bfloat16), jnp.uint16)
            lo = bits[:, : D // 2].astype(jnp.uint32)
            hi = bits[:, D // 2 :].astype(jnp.uint32)
            y_ref[r] = lax.bitcast_convert_type(lo | (hi << 16), jnp.int32)
        a12_ref[...] = (
            jnp.dot(xb16, wa_ref[...].astype(jnp.bfloat16),
                    preferred_element_type=jnp.float32)
            + ba_ref[...]
        )

    return pl.pallas_call(
        body,
        grid=(G,),
        in_specs=[
            pl.BlockSpec((BM, D), lambda i: (i, 0)),
            pl.BlockSpec((D, D), lambda i: (0, 0)),
            pl.BlockSpec((D,), lambda i: (0,)),
            pl.BlockSpec((R, D, D), lambda i: (0, 0, 0)),
            pl.BlockSpec((R, D), lambda i: (0, 0)),
            pl.BlockSpec((D, 2), lambda i: (0, 0)),
            pl.BlockSpec((1, 2), lambda i: (0, 0)),
        ],
        out_specs=[
            pl.BlockSpec((BM, D), lambda i: (i, 0)),
            pl.BlockSpec((R, BM, D // 2), lambda i: (0, i, 0)),
            pl.BlockSpec((BM, 2), lambda i: (i, 0)),
        ],
        out_shape=[
            jax.ShapeDtypeStruct((N, D), jnp.float32),
            jax.ShapeDtypeStruct((R, N, D // 2), jnp.int32),
            jax.ShapeDtypeStruct((N, 2), jnp.float32),
        ],
    )(x, W_self, b_self, W_rel, b_rel, wa, ba2)


SCN = 8
GRP = CW // SCN


def _sc_edge_agg(yflat, pk, srcp, typp, dstp2):
    mesh = plsc.VectorSubcoreMesh(core_axis_name="c", subcore_axis_name="s")

    @functools.partial(
        pl.kernel,
        out_type=jax.ShapeDtypeStruct((NC, ACC_ROWS, D), jnp.float32),
        mesh=mesh,
        compiler_params=pltpu.CompilerParams(
            needs_layout_passes=False, use_tc_tiling_on_sc=False),
        scratch_types=[
            pltpu.VMEM((NPAD,), jnp.int32),
            pltpu.VMEM((SCN, CH), jnp.int32),
            pltpu.VMEM((2 * SCN, CH // 2), jnp.int32),
            pltpu.VMEM((SCN, CH), jnp.float32),
            pltpu.VMEM((2 * SCN, CH // 2), jnp.int32),
            pltpu.VMEM((CH, D // 2), jnp.int32),
            pltpu.VMEM((CH, D // 2), jnp.int32),
            pltpu.VMEM((CH, D), jnp.float32),
            pltpu.VMEM_SHARED((ACC_ROWS, D), jnp.float32),
            pltpu.SemaphoreType.DMA,
            pltpu.SemaphoreType.DMA,
            pltpu.SemaphoreType.DMA,
            pltpu.SemaphoreType.DMA,
            pltpu.SemaphoreType.DMA,
            pltpu.SemaphoreType.DMA,
        ],
    )
    def k(y_hbm, pk_hbm, src_hbm, typ_hbm, dst2_hbm,
          part_hbm, pkv, srcv, ridv, attv, dstv2, rin0, rin1, rout, acc,
          semg0A, semg0B, semg1A, semg1B, semsA, semsB):
        c = lax.axis_index("c")
        s = lax.axis_index("s")
        w = s * NC + c

        pltpu.sync_copy(pk_hbm, pkv)

        z16 = jnp.zeros((16,), jnp.float32)

        def zrow(i, carry):
            def zcol(q, carry2):
                rout[i, pl.ds(q * 16, 16)] = z16
                return carry2
            return lax.fori_loop(0, D // 16, zcol, carry)
        lax.fori_loop(0, CH, zrow, 0)
        for m in range(RPT // CH):
            pltpu.sync_copy(rout, acc.at[pl.ds(s * RPT + m * CH, CH)])
        plsc.subcore_barrier()

        hi_mask = jnp.full((16,), -65536, jnp.int32)

        def group(g, carry):
            gsl = pl.ds(g * SCN, SCN)
            g2sl = pl.ds(g * 2 * SCN, 2 * SCN)
            pltpu.sync_copy(src_hbm.at[w, gsl], srcv)
            pltpu.sync_copy(typ_hbm.at[w, g2sl], ridv)
            pltpu.sync_copy(dst2_hbm.at[w, g2sl], dstv2)

            def att_row(j, carry1):
                def att_lane(q, carry2):
                    sl = pl.ds(q * 16, 16)
                    h2 = 2 * j + q // 4
                    sl2 = pl.ds((q % 4) * 16, 16)
                    s16 = srcv[j, sl]
                    d16 = dstv2[h2, sl2]
                    t16 = ridv[h2, sl2]
                    g1 = plsc.load_gather(pkv, [s16])
                    g2 = plsc.load_gather(pkv, [d16])
                    a1f = plsc.bitcast(lax.shift_left(g1, 16), jnp.float32)
                    a2f = plsc.bitcast(g2 & hi_mask, jnp.float32)
                    z = a1f + a2f
                    attv[j, sl] = 1.0 / (1.0 + jnp.exp(-z))
                    ridv[h2, sl2] = t16 * N + s16
                    return carry2
                return lax.fori_loop(0, CH // 16, att_lane, carry1)

            bufs = (rin0, rin1)
            sems = (semsA, semsB)
            HC = CH // 2

            def unpack_half(j, h, rin):
                def us16(gg, carry2):
                    att16 = attv[j, pl.ds(h * HC + gg * 16, 16)]
                    for l in range(16):
                        a = att16[l]
                        e = h * HC + gg * 16 + l
                        for q in range(D // 32):
                            gv = rin[e, pl.ds(q * 16, 16)]
                            lo = plsc.bitcast(lax.shift_left(gv, 16),
                                              jnp.float32)
                            hi = plsc.bitcast(gv & hi_mask, jnp.float32)
                            rout[e, pl.ds(q * 16, 16)] = lo * a
                            rout[e, pl.ds(D // 2 + q * 16, 16)] = hi * a
                    return carry2
                lax.fori_loop(0, HC // 16, us16, 0)

            semgs = ((semg0A, semg0B), (semg1A, semg1B))

            def gath(j, nb):
                return tuple(
                    pltpu.async_copy(
                        y_hbm.at[ridv.at[2 * j + h]],
                        bufs[nb].at[pl.ds(h * HC, HC)], semgs[nb][h])
                    for h in range(2))

            att_row(0, 0)
            descs_g = [None] * SCN
            descs_s = [[None] * SCN, [None] * SCN]
            descs_g[0] = gath(0, 0)
            lax.fori_loop(1, SCN, att_row, 0)
            for j in range(SCN):
                b = j % 2
                if j + 1 < SCN:
                    descs_g[j + 1] = gath(j + 1, (j + 1) % 2)
                for h in range(2):
                    descs_g[j][h].wait()
                    if j >= 1:
                        descs_s[h][j - 1].wait()
                    unpack_half(j, h, bufs[b])
                    descs_s[h][j] = pltpu.async_copy(
                        rout.at[pl.ds(h * HC, HC)],
                        acc.at[dstv2.at[2 * j + h]], sems[h], add=True)
            descs_s[0][SCN - 1].wait()
            descs_s[1][SCN - 1].wait()
            return carry
        lax.fori_loop(0, GRP, group, 0)

        plsc.subcore_barrier()
        sl = pl.ds(s * RPT, RPT)
        pltpu.sync_copy(acc.at[sl], part_hbm.at[c, sl])

    return k(yflat, pk, srcp, typp, dstp2)


def _combine(out0, part):
    BM = 400
    G = N // BM

    def body(o0_ref, p_ref, o_ref):
        o_ref[...] = o0_ref[...] + p_ref[0] + p_ref[1]

    return pl.pallas_call(
        body,
        grid=(G,),
        in_specs=[
            pl.BlockSpec((BM, D), lambda i: (i, 0)),
            pl.BlockSpec((NC, BM, D), lambda i: (0, i, 0)),
        ],
        out_specs=pl.BlockSpec((BM, D), lambda i: (i, 0)),
        out_shape=jax.ShapeDtypeStruct((N, D), jnp.float32),
    )(out0, part)


def kernel(x, edge_index, edge_types, W_self, b_self, W_rel, b_rel, W_att, b_att):
    wa = jnp.stack([W_att[:D, 0], W_att[D:, 0]], axis=1)
    ba2 = jnp.stack([b_att[0], jnp.zeros((), jnp.float32)])[None, :]

    out0, y, a12 = _dense_prep(x, W_self, b_self, W_rel, b_rel, wa, ba2)
    yflat = y.reshape(R * N, D // 2)
    a1p = jnp.pad(a12[:, 0], (0, NPAD - N))
    a2p = jnp.pad(a12[:, 1], (0, NPAD - N))
    b1 = lax.bitcast_convert_type(a1p.astype(jnp.bfloat16), jnp.uint16)
    b2 = lax.bitcast_convert_type(a2p.astype(jnp.bfloat16), jnp.uint16)
    pk = lax.bitcast_convert_type(
        b1.astype(jnp.uint32) | (b2.astype(jnp.uint32) << 16), jnp.int32)

    pad = E_PAD - E
    idt = edge_index.dtype
    srcp = jnp.concatenate(
        [edge_index[0], jnp.zeros((pad,), idt)]).reshape(NW, CW, CH)
    dstp = jnp.concatenate(
        [edge_index[1], jnp.full((pad,), N, idt)]).reshape(NW, CW, CH)
    typp = jnp.concatenate(
        [edge_types, jnp.zeros((pad,), edge_types.dtype)]
    ).reshape(NW, 2 * CW, CH // 2)

    dstp2 = dstp.reshape(NW, 2 * CW, CH // 2)
    part = _sc_edge_agg(yflat, pk, srcp, typp, dstp2)
    return _combine(out0, part)
